# baseline (speedup 1.0000x reference)
import functools

import jax
import jax.numpy as jnp
from jax import lax
from jax.experimental import pallas as pl
from jax.experimental.pallas import tpu as pltpu
from jax.experimental.pallas import tpu_sc as plsc

N, C, H, W = 4, 32, 384, 384
HW = H * W
NP = N * HW
NWORK = 32
PW = NP // NWORK
B = 288
NCHUNK = PW // B
L = 16


def _reflect_floor(v, size):
    span = float(size - 1)
    a = jnp.abs(v)
    extra = jnp.mod(a, span)
    flips = (a / span).astype(jnp.int32)
    r = jnp.where((flips & 1) == 0, extra, span - extra)
    r = jnp.clip(r, 0.0, span)
    i0 = r.astype(jnp.int32)
    frac = r - i0.astype(jnp.float32)
    return i0, frac


def _sc_body(table, gxh, gyh, outh,
             gx_v, gy_v, idx_v, wgt_v, rows_v, out_v,
             sem0, sem1, gsem0, gsem1, osem0, osem1):
    wid = lax.axis_index("s") * 2 + lax.axis_index("c")
    base_row = (wid // 8) * HW
    n = wid // 8
    sems = (sem0, sem1)
    gsems = (gsem0, gsem1)
    osems = (osem0, osem1)

    def fire_grid(b, t):
        p0 = wid * PW + t * B
        pltpu.async_copy(gxh.at[pl.ds(p0, B)], gx_v.at[b], gsems[b])
        pltpu.async_copy(gyh.at[pl.ds(p0, B)], gy_v.at[b], gsems[b])

    def drain_grid(b):
        pltpu.make_async_copy(gxh.at[pl.ds(0, B)], gx_v.at[b],
                              gsems[b]).wait()
        pltpu.make_async_copy(gyh.at[pl.ds(0, B)], gy_v.at[b],
                              gsems[b]).wait()

    def stage(b, t):
        @pl.when(t + 1 < NCHUNK)
        def _():
            fire_grid(1 - b, t + 1)

        drain_grid(b)

        @plsc.parallel_loop(0, B // L, unroll=2)
        def idx_body(g):
            sl = pl.ds(g * L, L)
            ix = (gx_v[b, sl] + 1.0) * 0.5 * (W - 1)
            iy = (gy_v[b, sl] + 1.0) * 0.5 * (H - 1)
            ix0, wx1 = _reflect_floor(ix, W)
            iy0, wy1 = _reflect_floor(iy, H)
            ix0c = jnp.minimum(ix0, W - 1)
            ix1c = jnp.minimum(ix0 + 1, W - 1)
            iy0c = jnp.minimum(iy0, H - 1)
            iy1c = jnp.minimum(iy0 + 1, H - 1)
            r0 = base_row + iy0c * W
            r1 = base_row + iy1c * W
            idx_v[b, pl.ds(0 * B + g * L, L)] = r0 + ix0c
            idx_v[b, pl.ds(1 * B + g * L, L)] = r0 + ix1c
            idx_v[b, pl.ds(2 * B + g * L, L)] = r1 + ix0c
            idx_v[b, pl.ds(3 * B + g * L, L)] = r1 + ix1c
            wx0 = 1.0 - wx1
            wy0 = 1.0 - wy1
            wgt_v[b, pl.ds(0 * B + g * L, L)] = wy0 * wx0
            wgt_v[b, pl.ds(1 * B + g * L, L)] = wy0 * wx1
            wgt_v[b, pl.ds(2 * B + g * L, L)] = wy1 * wx0
            wgt_v[b, pl.ds(3 * B + g * L, L)] = wy1 * wx1

        return pltpu.async_copy(table.at[idx_v.at[b]], rows_v.at[b], sems[b])

    chv = [lax.iota(jnp.int32, L) + h * L for h in range(C // L)]

    def out_slice(t):
        lp0 = (wid % 8) * PW + t * B
        return outh.at[n, :, pl.ds(lp0, B)]

    def blend(b, t):
        @pl.when(t >= 2)
        def _():
            pltpu.make_async_copy(out_v.at[b, :, pl.ds(0, B)],
                                  out_slice(t), osems[b]).wait()

        @plsc.parallel_loop(0, B // L, unroll=3)
        def blend_body(g):
            sl = pl.ds(g * L, L)
            w00v = wgt_v[b, pl.ds(0 * B + g * L, L)]
            w01v = wgt_v[b, pl.ds(1 * B + g * L, L)]
            w10v = wgt_v[b, pl.ds(2 * B + g * L, L)]
            w11v = wgt_v[b, pl.ds(3 * B + g * L, L)]
            pb = jnp.full((L,), g * L, dtype=jnp.int32)
            for i in range(L):
                ci = jnp.full((L,), i, dtype=jnp.int32)
                w00 = w00v.at[ci].get(mode="promise_in_bounds")
                w01 = w01v.at[ci].get(mode="promise_in_bounds")
                w10 = w10v.at[ci].get(mode="promise_in_bounds")
                w11 = w11v.at[ci].get(mode="promise_in_bounds")
                p = g * L + i
                for h in range(C // L):
                    cs = pl.ds(h * L, L)
                    acc = ((rows_v[b, 0 * B + p, cs] * w00 +
                            rows_v[b, 1 * B + p, cs] * w01) +
                           (rows_v[b, 2 * B + p, cs] * w10 +
                            rows_v[b, 3 * B + p, cs] * w11))
                    plsc.store_scatter(out_v.at[b], [chv[h], pb + i], acc)

        pltpu.async_copy(out_v.at[b, :, pl.ds(0, B)], out_slice(t), osems[b])

    fire_grid(0, 0)
    stage(0, 0)

    def outer_body(to, carry):
        for b in range(2):
            t = to * 2 + b

            @pl.when(t + 1 < NCHUNK)
            def _():
                stage(1 - b, t + 1)

            pltpu.make_async_copy(
                table.at[idx_v.at[b]], rows_v.at[b], sems[b]).wait()
            blend(b, t)
        return carry

    lax.fori_loop(0, NCHUNK // 2, outer_body, 0)

    pltpu.make_async_copy(out_v.at[0, :, pl.ds(0, B)],
                          out_slice(NCHUNK - 2), osems[0]).wait()
    pltpu.make_async_copy(out_v.at[1, :, pl.ds(0, B)],
                          out_slice(NCHUNK - 1), osems[1]).wait()


TB = 768
NTCH = PW // TB


def _tr_body(inp, tab, in_v, tr_v, sem0, sem1, osem0, osem1):
    wid = lax.axis_index("s") * 2 + lax.axis_index("c")
    n = wid // 8
    sems = (sem0, sem1)
    osems = (osem0, osem1)

    def fire(b, t):
        lp0 = (wid % 8) * PW + t * TB
        return pltpu.async_copy(inp.at[n, :, pl.ds(lp0, TB)], in_v.at[b],
                                sems[b])

    def tab_slice(t):
        gp0 = wid * PW + t * TB
        return tab.at[pl.ds(gp0, TB)]

    pix = lax.iota(jnp.int32, L)

    def transpose_chunk(b, t):
        pltpu.make_async_copy(
            inp.at[n, :, pl.ds(0, TB)], in_v.at[b], sems[b]).wait()

        @pl.when(t >= 2)
        def _():
            pltpu.make_async_copy(tr_v.at[b, :, pl.ds(0, C)],
                                  tab_slice(t), osems[b]).wait()

        @plsc.parallel_loop(0, C, unroll=2)
        def tr_loop(c):
            cv = jnp.full((L,), c, dtype=jnp.int32)
            for g in range(TB // L):
                v = in_v[b, c, pl.ds(g * L, L)]
                plsc.store_scatter(tr_v.at[b], [pix + g * L, cv], v)

        pltpu.async_copy(tr_v.at[b, :, pl.ds(0, C)], tab_slice(t), osems[b])

    fire(0, 0)

    def outer_body(to, carry):
        for b in range(2):
            t = to * 2 + b

            @pl.when(t + 1 < NTCH)
            def _():
                fire(1 - b, t + 1)

            transpose_chunk(b, t)
        return carry

    lax.fori_loop(0, NTCH // 2, outer_body, 0)
    pltpu.make_async_copy(tr_v.at[0, :, pl.ds(0, C)],
                          tab_slice(NTCH - 2), osems[0]).wait()
    pltpu.make_async_copy(tr_v.at[1, :, pl.ds(0, C)],
                          tab_slice(NTCH - 1), osems[1]).wait()


@jax.jit
def _transpose_sc(inp):
    mesh = plsc.VectorSubcoreMesh(core_axis_name="c", subcore_axis_name="s")
    kfn = functools.partial(
        pl.kernel,
        mesh=mesh,
        compiler_params=pltpu.CompilerParams(
            use_tc_tiling_on_sc=False, needs_layout_passes=False),
        out_type=jax.ShapeDtypeStruct((NP, C), jnp.float32),
        scratch_types=[
            pltpu.VMEM((2, C, TB), jnp.float32),
            pltpu.VMEM((2, TB, C + 1), jnp.float32),
            pltpu.SemaphoreType.DMA,
            pltpu.SemaphoreType.DMA,
            pltpu.SemaphoreType.DMA,
            pltpu.SemaphoreType.DMA,
        ],
    )(_tr_body)
    return kfn(inp)


@jax.jit
def _grid_sample_sc(tableT, gx, gy):
    mesh = plsc.VectorSubcoreMesh(core_axis_name="c", subcore_axis_name="s")
    kfn = functools.partial(
        pl.kernel,
        mesh=mesh,
        compiler_params=pltpu.CompilerParams(
            use_tc_tiling_on_sc=False, needs_layout_passes=False),
        out_type=jax.ShapeDtypeStruct((N, C, HW), jnp.float32),
        scratch_types=[
            pltpu.VMEM((2, B), jnp.float32),
            pltpu.VMEM((2, B), jnp.float32),
            pltpu.VMEM((2, 4 * B), jnp.int32),
            pltpu.VMEM((2, 4 * B), jnp.float32),
            pltpu.VMEM((2, 4 * B, C), jnp.float32),
            pltpu.VMEM((2, C, B + 1), jnp.float32),
            pltpu.SemaphoreType.DMA,
            pltpu.SemaphoreType.DMA,
            pltpu.SemaphoreType.DMA,
            pltpu.SemaphoreType.DMA,
            pltpu.SemaphoreType.DMA,
            pltpu.SemaphoreType.DMA,
        ],
    )(_sc_body)
    return kfn(tableT, gx, gy)


def kernel(input, grid):
    tableT = _transpose_sc(input.reshape(N, C, HW))
    gx = grid[..., 0].reshape(NP)
    gy = grid[..., 1].reshape(NP)
    out = _grid_sample_sc(tableT, gx, gy)
    return out.reshape(N, C, H, W)


# speedup vs baseline: 1.0274x; 1.0274x over previous
import functools

import jax
import jax.numpy as jnp
from jax import lax
from jax.experimental import pallas as pl
from jax.experimental.pallas import tpu as pltpu
from jax.experimental.pallas import tpu_sc as plsc

N, C, H, W = 4, 32, 384, 384
HW = H * W
NP = N * HW
NWORK = 32
PW = NP // NWORK
B = 288
NCHUNK = PW // B
L = 16


def _reflect_floor(v, size):
    span = float(size - 1)
    a = jnp.abs(v)
    extra = jnp.mod(a, span)
    flips = (a / span).astype(jnp.int32)
    r = jnp.where((flips & 1) == 0, extra, span - extra)
    r = jnp.clip(r, 0.0, span)
    i0 = r.astype(jnp.int32)
    frac = r - i0.astype(jnp.float32)
    return i0, frac


def _sc_body(table, gxh, gyh, outh,
             gx_v, gy_v, idx_v, wgt_v, rows_v, out_v,
             sem0, sem1, gsem0, gsem1, osem0, osem1):
    wid = lax.axis_index("s") * 2 + lax.axis_index("c")
    base_row = (wid // 8) * HW
    n = wid // 8
    sems = (sem0, sem1)
    gsems = (gsem0, gsem1)
    osems = (osem0, osem1)

    def fire_grid(b, t):
        p0 = wid * PW + t * B
        pltpu.async_copy(gxh.at[pl.ds(p0, B)], gx_v.at[b], gsems[b])
        pltpu.async_copy(gyh.at[pl.ds(p0, B)], gy_v.at[b], gsems[b])

    def drain_grid(b):
        pltpu.make_async_copy(gxh.at[pl.ds(0, B)], gx_v.at[b],
                              gsems[b]).wait()
        pltpu.make_async_copy(gyh.at[pl.ds(0, B)], gy_v.at[b],
                              gsems[b]).wait()

    def stage(b, t):
        @pl.when(t + 1 < NCHUNK)
        def _():
            fire_grid(1 - b, t + 1)

        drain_grid(b)

        @plsc.parallel_loop(0, B // L, unroll=3)
        def idx_body(g):
            sl = pl.ds(g * L, L)
            ix = (gx_v[b, sl] + 1.0) * 0.5 * (W - 1)
            iy = (gy_v[b, sl] + 1.0) * 0.5 * (H - 1)
            ix0, wx1 = _reflect_floor(ix, W)
            iy0, wy1 = _reflect_floor(iy, H)
            ix0c = jnp.minimum(ix0, W - 1)
            ix1c = jnp.minimum(ix0 + 1, W - 1)
            iy0c = jnp.minimum(iy0, H - 1)
            iy1c = jnp.minimum(iy0 + 1, H - 1)
            r0 = base_row + iy0c * W
            r1 = base_row + iy1c * W
            idx_v[b, pl.ds(0 * B + g * L, L)] = r0 + ix0c
            idx_v[b, pl.ds(1 * B + g * L, L)] = r0 + ix1c
            idx_v[b, pl.ds(2 * B + g * L, L)] = r1 + ix0c
            idx_v[b, pl.ds(3 * B + g * L, L)] = r1 + ix1c
            wx0 = 1.0 - wx1
            wy0 = 1.0 - wy1
            wgt_v[b, pl.ds(0 * B + g * L, L)] = wy0 * wx0
            wgt_v[b, pl.ds(1 * B + g * L, L)] = wy0 * wx1
            wgt_v[b, pl.ds(2 * B + g * L, L)] = wy1 * wx0
            wgt_v[b, pl.ds(3 * B + g * L, L)] = wy1 * wx1

        return pltpu.async_copy(table.at[idx_v.at[b]], rows_v.at[b], sems[b])

    chv = [lax.iota(jnp.int32, L) + h * L for h in range(C // L)]

    def out_slice(t):
        lp0 = (wid % 8) * PW + t * B
        return outh.at[n, :, pl.ds(lp0, B)]

    def blend(b, t):
        @pl.when(t >= 2)
        def _():
            pltpu.make_async_copy(out_v.at[b, :, pl.ds(0, B)],
                                  out_slice(t), osems[b]).wait()

        @plsc.parallel_loop(0, B // L, unroll=3)
        def blend_body(g):
            sl = pl.ds(g * L, L)
            w00v = wgt_v[b, pl.ds(0 * B + g * L, L)]
            w01v = wgt_v[b, pl.ds(1 * B + g * L, L)]
            w10v = wgt_v[b, pl.ds(2 * B + g * L, L)]
            w11v = wgt_v[b, pl.ds(3 * B + g * L, L)]
            pb = jnp.full((L,), g * L, dtype=jnp.int32)
            for i in range(L):
                ci = jnp.full((L,), i, dtype=jnp.int32)
                w00 = w00v.at[ci].get(mode="promise_in_bounds")
                w01 = w01v.at[ci].get(mode="promise_in_bounds")
                w10 = w10v.at[ci].get(mode="promise_in_bounds")
                w11 = w11v.at[ci].get(mode="promise_in_bounds")
                p = g * L + i
                for h in range(C // L):
                    cs = pl.ds(h * L, L)
                    acc = ((rows_v[b, 0 * B + p, cs] * w00 +
                            rows_v[b, 1 * B + p, cs] * w01) +
                           (rows_v[b, 2 * B + p, cs] * w10 +
                            rows_v[b, 3 * B + p, cs] * w11))
                    plsc.store_scatter(out_v.at[b], [chv[h], pb + i], acc)

        pltpu.async_copy(out_v.at[b, :, pl.ds(0, B)], out_slice(t), osems[b])

    fire_grid(0, 0)
    stage(0, 0)

    def outer_body(to, carry):
        for b in range(2):
            t = to * 2 + b

            @pl.when(t + 1 < NCHUNK)
            def _():
                stage(1 - b, t + 1)

            pltpu.make_async_copy(
                table.at[idx_v.at[b]], rows_v.at[b], sems[b]).wait()
            blend(b, t)
        return carry

    lax.fori_loop(0, NCHUNK // 2, outer_body, 0)

    pltpu.make_async_copy(out_v.at[0, :, pl.ds(0, B)],
                          out_slice(NCHUNK - 2), osems[0]).wait()
    pltpu.make_async_copy(out_v.at[1, :, pl.ds(0, B)],
                          out_slice(NCHUNK - 1), osems[1]).wait()


TB = 512
NTCH = PW // TB


def _tr_body(inp, tab, in_v, tr_v, sem0, sem1, osem0, osem1):
    wid = lax.axis_index("s") * 2 + lax.axis_index("c")
    n = wid // 8
    sems = (sem0, sem1)
    osems = (osem0, osem1)

    def fire(b, t):
        lp0 = (wid % 8) * PW + t * TB
        return pltpu.async_copy(inp.at[n, :, pl.ds(lp0, TB)], in_v.at[b],
                                sems[b])

    def tab_slice(t):
        gp0 = wid * PW + t * TB
        return tab.at[pl.ds(gp0, TB)]

    pix = lax.iota(jnp.int32, L)

    def transpose_chunk(b, t):
        pltpu.make_async_copy(
            inp.at[n, :, pl.ds(0, TB)], in_v.at[b], sems[b]).wait()

        @pl.when(t >= 2)
        def _():
            pltpu.make_async_copy(tr_v.at[b, :, pl.ds(0, C)],
                                  tab_slice(t), osems[b]).wait()

        @plsc.parallel_loop(0, C, unroll=4)
        def tr_loop(c):
            cv = jnp.full((L,), c, dtype=jnp.int32)
            for g in range(TB // L):
                v = in_v[b, c, pl.ds(g * L, L)]
                plsc.store_scatter(tr_v.at[b], [pix + g * L, cv], v)

        pltpu.async_copy(tr_v.at[b, :, pl.ds(0, C)], tab_slice(t), osems[b])

    fire(0, 0)

    def outer_body(to, carry):
        for b in range(2):
            t = to * 2 + b

            @pl.when(t + 1 < NTCH)
            def _():
                fire(1 - b, t + 1)

            transpose_chunk(b, t)
        return carry

    lax.fori_loop(0, NTCH // 2, outer_body, 0)
    pltpu.make_async_copy(tr_v.at[0, :, pl.ds(0, C)],
                          tab_slice(NTCH - 2), osems[0]).wait()
    pltpu.make_async_copy(tr_v.at[1, :, pl.ds(0, C)],
                          tab_slice(NTCH - 1), osems[1]).wait()


@jax.jit
def _transpose_sc(inp):
    mesh = plsc.VectorSubcoreMesh(core_axis_name="c", subcore_axis_name="s")
    kfn = functools.partial(
        pl.kernel,
        mesh=mesh,
        compiler_params=pltpu.CompilerParams(
            use_tc_tiling_on_sc=False, needs_layout_passes=False),
        out_type=jax.ShapeDtypeStruct((NP, C), jnp.float32),
        scratch_types=[
            pltpu.VMEM((2, C, TB), jnp.float32),
            pltpu.VMEM((2, TB, C + 1), jnp.float32),
            pltpu.SemaphoreType.DMA,
            pltpu.SemaphoreType.DMA,
            pltpu.SemaphoreType.DMA,
            pltpu.SemaphoreType.DMA,
        ],
    )(_tr_body)
    return kfn(inp)


@jax.jit
def _grid_sample_sc(tableT, gx, gy):
    mesh = plsc.VectorSubcoreMesh(core_axis_name="c", subcore_axis_name="s")
    kfn = functools.partial(
        pl.kernel,
        mesh=mesh,
        compiler_params=pltpu.CompilerParams(
            use_tc_tiling_on_sc=False, needs_layout_passes=False),
        out_type=jax.ShapeDtypeStruct((N, C, HW), jnp.float32),
        scratch_types=[
            pltpu.VMEM((2, B), jnp.float32),
            pltpu.VMEM((2, B), jnp.float32),
            pltpu.VMEM((2, 4 * B), jnp.int32),
            pltpu.VMEM((2, 4 * B), jnp.float32),
            pltpu.VMEM((2, 4 * B, C), jnp.float32),
            pltpu.VMEM((2, C, B + 1), jnp.float32),
            pltpu.SemaphoreType.DMA,
            pltpu.SemaphoreType.DMA,
            pltpu.SemaphoreType.DMA,
            pltpu.SemaphoreType.DMA,
            pltpu.SemaphoreType.DMA,
            pltpu.SemaphoreType.DMA,
        ],
    )(_sc_body)
    return kfn(tableT, gx, gy)


def kernel(input, grid):
    tableT = _transpose_sc(input.reshape(N, C, HW))
    gx = grid[..., 0].reshape(NP)
    gy = grid[..., 1].reshape(NP)
    out = _grid_sample_sc(tableT, gx, gy)
    return out.reshape(N, C, H, W)
